# baseline (speedup 1.0000x reference)
import functools

import numpy as np
import jax
import jax.numpy as jnp
from jax.experimental import pallas as pl
from jax.experimental.pallas import tpu as pltpu

_L = 200
_D = 128
_RATIO = 0.25
_BB = 64


def _block_lists(batch):
    num = int(batch * _RATIO)
    with jax.ensure_compile_time_eval():
        perm = np.asarray(jax.random.permutation(jax.random.key(1), batch))
    ind = np.sort(perm[:num])
    nblocks = batch // _BB
    lists = []
    for b in range(nblocks):
        local = [int(r - b * _BB) for r in ind if b * _BB <= r < (b + 1) * _BB]
        lists.append(local)
    maxp = max(len(l) for l in lists)
    padded = np.array(
        [l + [l[0]] * (maxp - len(l)) for l in lists], dtype=np.int32
    )
    return padded, maxp


def _body(plist_ref, x_ref, o_ref, *, maxp):
    i = pl.program_id(0)
    o_ref[...] = x_ref[...]
    even = (jax.lax.broadcasted_iota(jnp.int32, (_L, _D), 0) % 2) == 0
    for k in range(maxp):
        j = plist_ref[i * maxp + k]
        x = x_ref[j]
        xm1 = jnp.concatenate([x[:1], x[:-1]], axis=0)
        avg = 0.5 * (x + xm1)
        avg_sh = jnp.concatenate([avg[:1], avg[:-1]], axis=0)
        o_ref[j] = jnp.where(even, avg, avg_sh)


def kernel(feats, max_len):
    batch = feats.shape[0]
    padded, maxp = _block_lists(batch)
    plist = jnp.asarray(padded.reshape(-1))

    grid_spec = pltpu.PrefetchScalarGridSpec(
        num_scalar_prefetch=1,
        grid=(batch // _BB,),
        in_specs=[pl.BlockSpec((_BB, _L, _D), lambda i, plist: (i, 0, 0))],
        out_specs=pl.BlockSpec((_BB, _L, _D), lambda i, plist: (i, 0, 0)),
    )
    return pl.pallas_call(
        functools.partial(_body, maxp=maxp),
        grid_spec=grid_spec,
        out_shape=jax.ShapeDtypeStruct(feats.shape, feats.dtype),
        compiler_params=pltpu.CompilerParams(
            dimension_semantics=("parallel",),
        ),
    )(plist, feats)
